# Initial kernel scaffold; baseline (speedup 1.0000x reference)
import jax
import jax.numpy as jnp
from jax.experimental import pallas as pl


def kernel(inputs, edge_index, W_self1, W_neigh1, b1, W_self2, W_neigh2, b2):
    raise NotImplementedError("write your pallas kernel here")


# speedup vs baseline: 5.3490x; 5.3490x over previous
import functools

import jax
import jax.numpy as jnp
from jax import lax
from jax.experimental import pallas as pl
from jax.experimental.pallas import tpu as pltpu
from jax.experimental.pallas import tpu_sc as plsc

N = 10000
E = 320000
D_IN = 128
D1 = 144
D2 = 48

NC = 2
NS = 16
NW = NC * NS
EPW = E // NW
CHUNK = 80
NCHUNK = EPW // CHUNK
N_PAD = 10240
RPT = N_PAD // NS
ZR = 40


def _make_edge_agg(d):
    mesh = plsc.VectorSubcoreMesh(core_axis_name="c", subcore_axis_name="s")

    @functools.partial(
        pl.kernel,
        mesh=mesh,
        out_type=jax.ShapeDtypeStruct((NC, N_PAD, d), jnp.float32),
        scratch_types=[
            pltpu.VMEM((CHUNK,), jnp.int32),
            pltpu.VMEM((CHUNK,), jnp.int32),
            pltpu.VMEM((CHUNK, d), jnp.float32),
            pltpu.VMEM((ZR, d), jnp.float32),
            pltpu.VMEM_SHARED((N_PAD, d), jnp.float32),
            pltpu.SemaphoreType.DMA,
        ],
        compiler_params=pltpu.CompilerParams(use_tc_tiling_on_sc=False),
    )
    def agg(t_hbm, src_hbm, dst_hbm, zeros_hbm, out_hbm,
            src_v, dst_v, rows_v, zbuf_v, acc, sem):
        c = lax.axis_index("c")
        s = lax.axis_index("s")
        w = c * NS + s

        pltpu.sync_copy(zeros_hbm, zbuf_v)

        def zstep(j, carry):
            pltpu.sync_copy(zbuf_v, acc.at[pl.ds(s * RPT + j * ZR, ZR)])
            return carry

        lax.fori_loop(0, RPT // ZR, zstep, 0)
        plsc.subcore_barrier()

        def step(i, carry):
            base = w * EPW + i * CHUNK
            pltpu.sync_copy(src_hbm.at[pl.ds(base, CHUNK)], src_v)
            pltpu.sync_copy(dst_hbm.at[pl.ds(base, CHUNK)], dst_v)
            pltpu.async_copy(t_hbm.at[src_v], rows_v, sem).wait()
            pltpu.sync_copy(rows_v, acc.at[dst_v], add=True)
            return carry

        lax.fori_loop(0, NCHUNK, step, 0)
        plsc.subcore_barrier()
        pltpu.sync_copy(acc.at[pl.ds(s * RPT, RPT)],
                        out_hbm.at[c, pl.ds(s * RPT, RPT)])

    return agg


_agg_d1 = _make_edge_agg(D1)
_agg_d2 = _make_edge_agg(D2)

_BN = 1000


def _tc_pre(x, wn1p):
    def body(x_ref, w_ref, o_ref):
        m = jnp.dot(x_ref[...], w_ref[...], preferred_element_type=jnp.float32)
        col = lax.broadcasted_iota(jnp.int32, (_BN, D1), 1)
        o_ref[...] = m + jnp.where(col == D_IN, 1.0, 0.0).astype(jnp.float32)

    return pl.pallas_call(
        body,
        grid=(N // _BN,),
        in_specs=[
            pl.BlockSpec((_BN, D_IN), lambda i: (i, 0)),
            pl.BlockSpec((D_IN, D1), lambda i: (0, 0)),
        ],
        out_specs=pl.BlockSpec((_BN, D1), lambda i: (i, 0)),
        out_shape=jax.ShapeDtypeStruct((N, D1), jnp.float32),
    )(x, wn1p)


def _tc_mid(x, p, ws1, b1r, wn2p, ws2):
    def body(x_ref, p_ref, ws1_ref, b1_ref, wn2_ref, ws2_ref,
             h1_ref, t2_ref, s2_ref, invd_ref):
        agg = p_ref[0] + p_ref[1]
        deg = agg[:, D_IN:D_IN + 1]
        invd = 1.0 / jnp.maximum(deg, 1.0)
        mean = agg[:, :D_IN] * invd
        h1 = jnp.maximum(
            jnp.dot(x_ref[...], ws1_ref[...],
                    preferred_element_type=jnp.float32) + mean + b1_ref[...],
            0.0)
        h1_ref[...] = h1
        t2_ref[...] = jnp.dot(h1, wn2_ref[...],
                              preferred_element_type=jnp.float32)
        s2_ref[...] = jnp.dot(h1, ws2_ref[...],
                              preferred_element_type=jnp.float32)
        invd_ref[...] = invd

    return pl.pallas_call(
        body,
        grid=(N // _BN,),
        in_specs=[
            pl.BlockSpec((_BN, D_IN), lambda i: (i, 0)),
            pl.BlockSpec((NC, _BN, D1), lambda i: (0, i, 0)),
            pl.BlockSpec((D_IN, D_IN), lambda i: (0, 0)),
            pl.BlockSpec((1, D_IN), lambda i: (0, 0)),
            pl.BlockSpec((D_IN, D2), lambda i: (0, 0)),
            pl.BlockSpec((D_IN, 40), lambda i: (0, 0)),
        ],
        out_specs=[
            pl.BlockSpec((_BN, D_IN), lambda i: (i, 0)),
            pl.BlockSpec((_BN, D2), lambda i: (i, 0)),
            pl.BlockSpec((_BN, 40), lambda i: (i, 0)),
            pl.BlockSpec((_BN, 1), lambda i: (i, 0)),
        ],
        out_shape=[
            jax.ShapeDtypeStruct((N, D_IN), jnp.float32),
            jax.ShapeDtypeStruct((N, D2), jnp.float32),
            jax.ShapeDtypeStruct((N, 40), jnp.float32),
            jax.ShapeDtypeStruct((N, 1), jnp.float32),
        ],
    )(x, p, ws1, b1r, wn2p, ws2)


def _tc_post(s2, q, invd, b2r):
    def body(s2_ref, q_ref, invd_ref, b2_ref, o_ref):
        aggq = q_ref[0] + q_ref[1]
        o_ref[...] = s2_ref[...] + aggq[:, :40] * invd_ref[...] + b2_ref[...]

    return pl.pallas_call(
        body,
        grid=(N // _BN,),
        in_specs=[
            pl.BlockSpec((_BN, 40), lambda i: (i, 0)),
            pl.BlockSpec((NC, _BN, D2), lambda i: (0, i, 0)),
            pl.BlockSpec((_BN, 1), lambda i: (i, 0)),
            pl.BlockSpec((1, 40), lambda i: (0, 0)),
        ],
        out_specs=pl.BlockSpec((_BN, 40), lambda i: (i, 0)),
        out_shape=jax.ShapeDtypeStruct((N, 40), jnp.float32),
    )(s2, q, invd, b2r)


def kernel(inputs, edge_index, W_self1, W_neigh1, b1, W_self2, W_neigh2, b2):
    src = edge_index[0]
    dst = edge_index[1]

    wn1p = jnp.pad(W_neigh1, ((0, 0), (0, D1 - D_IN)))
    wn2p = jnp.pad(W_neigh2, ((0, 0), (0, D2 - 40)))
    b1r = b1.reshape(1, D_IN)
    b2r = b2.reshape(1, 40)
    z1 = jnp.zeros((ZR, D1), jnp.float32)
    z2 = jnp.zeros((ZR, D2), jnp.float32)

    t1 = _tc_pre(inputs, wn1p)
    p = _agg_d1(t1, src, dst, z1)[:, :N]
    h1, t2p, s2, invd = _tc_mid(inputs, p, W_self1, b1r, wn2p, W_self2)
    q = _agg_d2(t2p, src, dst, z2)[:, :N]
    out = _tc_post(s2, q, invd, b2r)
    return (out, h1, out, h1)


# speedup vs baseline: 8.6929x; 1.6251x over previous
import functools

import jax
import jax.numpy as jnp
from jax import lax
from jax.experimental import pallas as pl
from jax.experimental.pallas import tpu as pltpu
from jax.experimental.pallas import tpu_sc as plsc

N = 10000
E = 320000
D_IN = 128
D1 = 144
D2 = 48

NC = 2
NS = 16
NW = NC * NS
EPW = E // NW
CHUNK = 80
NCHUNK = EPW // CHUNK
SUB = 25
NBLK = NCHUNK // SUB
N_PAD = 10240
RPT = N_PAD // NS
ZR = 40


def _make_edge_agg(d):
    mesh = plsc.VectorSubcoreMesh(core_axis_name="c", subcore_axis_name="s")

    @functools.partial(
        pl.kernel,
        mesh=mesh,
        out_type=jax.ShapeDtypeStruct((NC, N_PAD, d), jnp.float32),
        scratch_types=[
            pltpu.VMEM((SUB, CHUNK), jnp.int32),
            pltpu.VMEM((SUB, CHUNK), jnp.int32),
            pltpu.VMEM((2, CHUNK, d), jnp.float32),
            pltpu.VMEM((ZR, d), jnp.float32),
            pltpu.VMEM_SHARED((N_PAD, d), jnp.float32),
            pltpu.SemaphoreType.DMA,
            pltpu.SemaphoreType.DMA,
        ],
        compiler_params=pltpu.CompilerParams(use_tc_tiling_on_sc=False),
    )
    def agg(t_hbm, src_hbm, dst_hbm, zeros_hbm, out_hbm,
            srcb_v, dstb_v, rows_v, zbuf_v, acc, sem_s, sem_g):
        c = lax.axis_index("c")
        s = lax.axis_index("s")
        w = c * NS + s

        pltpu.sync_copy(zeros_hbm, zbuf_v)

        def zstep(j, carry):
            pltpu.sync_copy(zbuf_v, acc.at[pl.ds(s * RPT + j * ZR, ZR)])
            return carry

        lax.fori_loop(0, RPT // ZR, zstep, 0)
        plsc.subcore_barrier()

        chunk0 = w * NCHUNK

        def block(b, carry):
            row0 = chunk0 + b * SUB
            pltpu.sync_copy(src_hbm.at[pl.ds(row0, SUB)], srcb_v)
            pltpu.sync_copy(dst_hbm.at[pl.ds(row0, SUB)], dstb_v)
            pltpu.async_copy(t_hbm.at[srcb_v.at[0]], rows_v.at[0],
                             sem_g).wait()

            def step(j, carry2):
                p = lax.rem(j, 2)
                d_s = pltpu.async_copy(rows_v.at[p], acc.at[dstb_v.at[j]],
                                       sem_s, add=True)

                @pl.when(j < SUB - 1)
                def _():
                    pltpu.async_copy(t_hbm.at[srcb_v.at[j + 1]],
                                     rows_v.at[1 - p], sem_g).wait()

                d_s.wait()
                return carry2

            lax.fori_loop(0, SUB, step, 0)
            return carry

        lax.fori_loop(0, NBLK, block, 0)
        plsc.subcore_barrier()
        pltpu.sync_copy(acc.at[pl.ds(s * RPT, RPT)],
                        out_hbm.at[c, pl.ds(s * RPT, RPT)])

    return agg


_agg_d1 = _make_edge_agg(D1)
_agg_d2 = _make_edge_agg(D2)

_BN = 1000


def _tc_pre(x, wn1p):
    def body(x_ref, w_ref, o_ref):
        m = jnp.dot(x_ref[...], w_ref[...], preferred_element_type=jnp.float32)
        col = lax.broadcasted_iota(jnp.int32, (_BN, D1), 1)
        o_ref[...] = m + jnp.where(col == D_IN, 1.0, 0.0).astype(jnp.float32)

    return pl.pallas_call(
        body,
        grid=(N // _BN,),
        in_specs=[
            pl.BlockSpec((_BN, D_IN), lambda i: (i, 0)),
            pl.BlockSpec((D_IN, D1), lambda i: (0, 0)),
        ],
        out_specs=pl.BlockSpec((_BN, D1), lambda i: (i, 0)),
        out_shape=jax.ShapeDtypeStruct((N, D1), jnp.float32),
    )(x, wn1p)


def _tc_mid(x, p, ws1, b1r, wn2p, ws2):
    def body(x_ref, p_ref, ws1_ref, b1_ref, wn2_ref, ws2_ref,
             h1_ref, t2_ref, s2_ref, invd_ref):
        agg = p_ref[0] + p_ref[1]
        deg = agg[:, D_IN:D_IN + 1]
        invd = 1.0 / jnp.maximum(deg, 1.0)
        mean = agg[:, :D_IN] * invd
        h1 = jnp.maximum(
            jnp.dot(x_ref[...], ws1_ref[...],
                    preferred_element_type=jnp.float32) + mean + b1_ref[...],
            0.0)
        h1_ref[...] = h1
        t2_ref[...] = jnp.dot(h1, wn2_ref[...],
                              preferred_element_type=jnp.float32)
        s2_ref[...] = jnp.dot(h1, ws2_ref[...],
                              preferred_element_type=jnp.float32)
        invd_ref[...] = invd

    return pl.pallas_call(
        body,
        grid=(N // _BN,),
        in_specs=[
            pl.BlockSpec((_BN, D_IN), lambda i: (i, 0)),
            pl.BlockSpec((NC, _BN, D1), lambda i: (0, i, 0)),
            pl.BlockSpec((D_IN, D_IN), lambda i: (0, 0)),
            pl.BlockSpec((1, D_IN), lambda i: (0, 0)),
            pl.BlockSpec((D_IN, D2), lambda i: (0, 0)),
            pl.BlockSpec((D_IN, 40), lambda i: (0, 0)),
        ],
        out_specs=[
            pl.BlockSpec((_BN, D_IN), lambda i: (i, 0)),
            pl.BlockSpec((_BN, D2), lambda i: (i, 0)),
            pl.BlockSpec((_BN, 40), lambda i: (i, 0)),
            pl.BlockSpec((_BN, 1), lambda i: (i, 0)),
        ],
        out_shape=[
            jax.ShapeDtypeStruct((N, D_IN), jnp.float32),
            jax.ShapeDtypeStruct((N, D2), jnp.float32),
            jax.ShapeDtypeStruct((N, 40), jnp.float32),
            jax.ShapeDtypeStruct((N, 1), jnp.float32),
        ],
    )(x, p, ws1, b1r, wn2p, ws2)


def _tc_post(s2, q, invd, b2r):
    def body(s2_ref, q_ref, invd_ref, b2_ref, o_ref):
        aggq = q_ref[0] + q_ref[1]
        o_ref[...] = s2_ref[...] + aggq[:, :40] * invd_ref[...] + b2_ref[...]

    return pl.pallas_call(
        body,
        grid=(N // _BN,),
        in_specs=[
            pl.BlockSpec((_BN, 40), lambda i: (i, 0)),
            pl.BlockSpec((NC, _BN, D2), lambda i: (0, i, 0)),
            pl.BlockSpec((_BN, 1), lambda i: (i, 0)),
            pl.BlockSpec((1, 40), lambda i: (0, 0)),
        ],
        out_specs=pl.BlockSpec((_BN, 40), lambda i: (i, 0)),
        out_shape=jax.ShapeDtypeStruct((N, 40), jnp.float32),
    )(s2, q, invd, b2r)


def kernel(inputs, edge_index, W_self1, W_neigh1, b1, W_self2, W_neigh2, b2):
    src = edge_index[0].reshape(E // CHUNK, CHUNK)
    dst = edge_index[1].reshape(E // CHUNK, CHUNK)

    wn1p = jnp.pad(W_neigh1, ((0, 0), (0, D1 - D_IN)))
    wn2p = jnp.pad(W_neigh2, ((0, 0), (0, D2 - 40)))
    b1r = b1.reshape(1, D_IN)
    b2r = b2.reshape(1, 40)
    z1 = jnp.zeros((ZR, D1), jnp.float32)
    z2 = jnp.zeros((ZR, D2), jnp.float32)

    t1 = _tc_pre(inputs, wn1p)
    p = _agg_d1(t1, src, dst, z1)[:, :N]
    h1, t2p, s2, invd = _tc_mid(inputs, p, W_self1, b1r, wn2p, W_self2)
    q = _agg_d2(t2p, src, dst, z2)[:, :N]
    out = _tc_post(s2, q, invd, b2r)
    return (out, h1, out, h1)


# speedup vs baseline: 12.3672x; 1.4227x over previous
import functools

import jax
import jax.numpy as jnp
from jax import lax
from jax.experimental import pallas as pl
from jax.experimental.pallas import tpu as pltpu
from jax.experimental.pallas import tpu_sc as plsc

N = 10000
E = 320000
D_IN = 128
D1 = 144
D2 = 48

NC = 2
NS = 16
NW = NC * NS
EPW = E // NW
CHUNK = 80
NCHUNK = EPW // CHUNK
SUB = 25
NBLK = NCHUNK // SUB
RING = 3
NPT = N // NS


def _make_edge_agg(d, stage_t):
    mesh = plsc.VectorSubcoreMesh(core_axis_name="c", subcore_axis_name="s")

    scratch = [
        pltpu.VMEM((SUB, CHUNK), jnp.int32),
        pltpu.VMEM((SUB, CHUNK), jnp.int32),
        pltpu.VMEM((RING, CHUNK, d), jnp.float32),
        pltpu.VMEM_SHARED((N, d), jnp.float32),
        pltpu.VMEM_SHARED((N, d) if stage_t else (8, d), jnp.float32),
        pltpu.SemaphoreType.DMA,
        pltpu.SemaphoreType.DMA,
    ]

    @functools.partial(
        pl.kernel,
        mesh=mesh,
        out_type=jax.ShapeDtypeStruct((NC, N, d), jnp.float32),
        scratch_types=scratch,
        compiler_params=pltpu.CompilerParams(use_tc_tiling_on_sc=False),
    )
    def agg(t_hbm, src_hbm, dst_hbm, zeros_hbm, out_hbm,
            srcb_v, dstb_v, rows_v, acc, tstage, sem_s, sem_g):
        c = lax.axis_index("c")
        s = lax.axis_index("s")
        w = c * NS + s

        pltpu.sync_copy(zeros_hbm, acc.at[pl.ds(s * NPT, NPT)])
        if stage_t:
            pltpu.sync_copy(t_hbm.at[pl.ds(s * NPT, NPT)],
                            tstage.at[pl.ds(s * NPT, NPT)])
        gsrc = tstage if stage_t else t_hbm
        plsc.subcore_barrier()

        def gissue(j):
            pltpu.async_copy(gsrc.at[srcb_v.at[j]],
                             rows_v.at[lax.rem(j, RING)], sem_g)

        def gwait():
            pltpu.make_async_copy(gsrc.at[srcb_v.at[0]], rows_v.at[0],
                                  sem_g).wait()

        def swait():
            pltpu.make_async_copy(rows_v.at[0], acc.at[dstb_v.at[0]],
                                  sem_s).wait()

        chunk0 = w * NCHUNK

        def block(b, carry):
            row0 = chunk0 + b * SUB
            pltpu.sync_copy(src_hbm.at[pl.ds(row0, SUB)], srcb_v)
            pltpu.sync_copy(dst_hbm.at[pl.ds(row0, SUB)], dstb_v)
            gissue(0)
            gissue(1)

            def step(j, carry2):
                gwait()
                pltpu.async_copy(rows_v.at[lax.rem(j, RING)],
                                 acc.at[dstb_v.at[j]], sem_s, add=True)

                @pl.when(j + 2 < SUB)
                def _():
                    @pl.when(j >= 1)
                    def _():
                        swait()

                    gissue(j + 2)

                return carry2

            lax.fori_loop(0, SUB, step, 0)
            swait()
            swait()
            swait()
            return carry

        lax.fori_loop(0, NBLK, block, 0)
        plsc.subcore_barrier()
        pltpu.sync_copy(acc.at[pl.ds(s * NPT, NPT)],
                        out_hbm.at[c, pl.ds(s * NPT, NPT)])

    return agg


_agg_d1 = _make_edge_agg(D1, stage_t=False)
_agg_d2 = _make_edge_agg(D2, stage_t=True)

_BN = 1000


def _tc_pre(x, wn1p):
    def body(x_ref, w_ref, o_ref):
        m = jnp.dot(x_ref[...], w_ref[...], preferred_element_type=jnp.float32)
        col = lax.broadcasted_iota(jnp.int32, (_BN, D1), 1)
        o_ref[...] = m + jnp.where(col == D_IN, 1.0, 0.0).astype(jnp.float32)

    return pl.pallas_call(
        body,
        grid=(N // _BN,),
        in_specs=[
            pl.BlockSpec((_BN, D_IN), lambda i: (i, 0)),
            pl.BlockSpec((D_IN, D1), lambda i: (0, 0)),
        ],
        out_specs=pl.BlockSpec((_BN, D1), lambda i: (i, 0)),
        out_shape=jax.ShapeDtypeStruct((N, D1), jnp.float32),
    )(x, wn1p)


def _tc_mid(x, p, ws1, b1r, wn2p, ws2):
    def body(x_ref, p_ref, ws1_ref, b1_ref, wn2_ref, ws2_ref,
             h1_ref, t2_ref, s2_ref, invd_ref):
        agg = p_ref[0] + p_ref[1]
        deg = agg[:, D_IN:D_IN + 1]
        invd = 1.0 / jnp.maximum(deg, 1.0)
        mean = agg[:, :D_IN] * invd
        h1 = jnp.maximum(
            jnp.dot(x_ref[...], ws1_ref[...],
                    preferred_element_type=jnp.float32) + mean + b1_ref[...],
            0.0)
        h1_ref[...] = h1
        t2_ref[...] = jnp.dot(h1, wn2_ref[...],
                              preferred_element_type=jnp.float32)
        s2_ref[...] = jnp.dot(h1, ws2_ref[...],
                              preferred_element_type=jnp.float32)
        invd_ref[...] = invd

    return pl.pallas_call(
        body,
        grid=(N // _BN,),
        in_specs=[
            pl.BlockSpec((_BN, D_IN), lambda i: (i, 0)),
            pl.BlockSpec((NC, _BN, D1), lambda i: (0, i, 0)),
            pl.BlockSpec((D_IN, D_IN), lambda i: (0, 0)),
            pl.BlockSpec((1, D_IN), lambda i: (0, 0)),
            pl.BlockSpec((D_IN, D2), lambda i: (0, 0)),
            pl.BlockSpec((D_IN, 40), lambda i: (0, 0)),
        ],
        out_specs=[
            pl.BlockSpec((_BN, D_IN), lambda i: (i, 0)),
            pl.BlockSpec((_BN, D2), lambda i: (i, 0)),
            pl.BlockSpec((_BN, 40), lambda i: (i, 0)),
            pl.BlockSpec((_BN, 1), lambda i: (i, 0)),
        ],
        out_shape=[
            jax.ShapeDtypeStruct((N, D_IN), jnp.float32),
            jax.ShapeDtypeStruct((N, D2), jnp.float32),
            jax.ShapeDtypeStruct((N, 40), jnp.float32),
            jax.ShapeDtypeStruct((N, 1), jnp.float32),
        ],
    )(x, p, ws1, b1r, wn2p, ws2)


def _tc_post(s2, q, invd, b2r):
    def body(s2_ref, q_ref, invd_ref, b2_ref, o_ref):
        aggq = q_ref[0] + q_ref[1]
        o_ref[...] = s2_ref[...] + aggq[:, :40] * invd_ref[...] + b2_ref[...]

    return pl.pallas_call(
        body,
        grid=(N // _BN,),
        in_specs=[
            pl.BlockSpec((_BN, 40), lambda i: (i, 0)),
            pl.BlockSpec((NC, _BN, D2), lambda i: (0, i, 0)),
            pl.BlockSpec((_BN, 1), lambda i: (i, 0)),
            pl.BlockSpec((1, 40), lambda i: (0, 0)),
        ],
        out_specs=pl.BlockSpec((_BN, 40), lambda i: (i, 0)),
        out_shape=jax.ShapeDtypeStruct((N, 40), jnp.float32),
    )(s2, q, invd, b2r)


def kernel(inputs, edge_index, W_self1, W_neigh1, b1, W_self2, W_neigh2, b2):
    src = edge_index[0].reshape(E // CHUNK, CHUNK)
    dst = edge_index[1].reshape(E // CHUNK, CHUNK)

    wn1p = jnp.pad(W_neigh1, ((0, 0), (0, D1 - D_IN)))
    wn2p = jnp.pad(W_neigh2, ((0, 0), (0, D2 - 40)))
    b1r = b1.reshape(1, D_IN)
    b2r = b2.reshape(1, 40)
    z1 = jnp.zeros((NPT, D1), jnp.float32)
    z2 = jnp.zeros((NPT, D2), jnp.float32)

    t1 = _tc_pre(inputs, wn1p)
    p = _agg_d1(t1, src, dst, z1)
    h1, t2p, s2, invd = _tc_mid(inputs, p, W_self1, b1r, wn2p, W_self2)
    q = _agg_d2(t2p, src, dst, z2)
    out = _tc_post(s2, q, invd, b2r)
    return (out, h1, out, h1)
